# baseline (speedup 1.0000x reference)
import functools

import jax
import jax.numpy as jnp
from jax import lax
from jax.experimental import pallas as pl
from jax.experimental.pallas import tpu as pltpu
from jax.experimental.pallas import tpu_sc as plsc

_LANES = 16
_NBUF = 3


@functools.partial(jax.jit, static_argnums=(2, 3, 4, 5))
def _gather_mean(idx3, table, nb, nchunks, rpc, s):
    info = plsc.get_sparse_core_info()
    nc, ns = info.num_cores, info.num_subcores
    nw = nc * ns
    d = table.shape[1]
    bp = nw * nb

    mesh = plsc.VectorSubcoreMesh(core_axis_name="c", subcore_axis_name="s")

    @functools.partial(
        pl.kernel,
        mesh=mesh,
        out_type=jax.ShapeDtypeStruct((bp, d), jnp.float32),
        scratch_types=[
            pltpu.VMEM((nchunks, rpc * s), jnp.int32),
        ] + [pltpu.VMEM((rpc * s, d), jnp.float32)] * _NBUF + [
            pltpu.VMEM((nb, d), jnp.float32),
        ] + [pltpu.SemaphoreType.DMA] * _NBUF,
    )
    def k(idx_hbm, table_hbm, out_hbm, idx_v, *rest):
        bufs = rest[:_NBUF]
        out_v = rest[_NBUF]
        sems = rest[_NBUF + 1:]
        w = lax.axis_index("s") * nc + lax.axis_index("c")
        pltpu.sync_copy(idx_hbm.at[w], idx_v)

        scale = jnp.float32(1.0 / s)
        nvec = d // _LANES

        def compute(c, buf):
            for r in range(rpc):
                base = r * s
                accs = [buf[base, pl.ds(v * _LANES, _LANES)]
                        for v in range(nvec)]
                for j in range(1, s):
                    for v in range(nvec):
                        accs[v] = accs[v] + buf[base + j,
                                                pl.ds(v * _LANES, _LANES)]
                row = c * rpc + r
                for v in range(nvec):
                    out_v[row, pl.ds(v * _LANES, _LANES)] = accs[v] * scale

        for b in range(_NBUF - 1):
            pltpu.async_copy(table_hbm.at[idx_v.at[jnp.int32(b)]],
                             bufs[b], sems[b])

        def body(t, carry):
            for b in range(_NBUF):
                c = t * _NBUF + b
                pltpu.make_async_copy(table_hbm.at[idx_v.at[c]],
                                      bufs[b], sems[b]).wait()
                nxt = c + _NBUF - 1
                nb_slot = (b + _NBUF - 1) % _NBUF

                @pl.when(nxt < nchunks)
                def _():
                    pltpu.async_copy(table_hbm.at[idx_v.at[nxt]],
                                     bufs[nb_slot], sems[nb_slot])

                compute(c, bufs[b])
            return carry

        lax.fori_loop(jnp.int32(0), jnp.int32(nchunks // _NBUF), body,
                      jnp.int32(0))
        pltpu.sync_copy(out_v, out_hbm.at[pl.ds(w * nb, nb)])

    return k(idx3, table)


def kernel(nodes, neighbors, num_sample, emb_weight):
    b, s = neighbors.shape
    idx = neighbors.astype(jnp.int32)

    info = plsc.get_sparse_core_info()
    nw = info.num_cores * info.num_subcores
    rpc = max(1, 128 // s)
    align = nw * rpc * _NBUF
    while align % (nw * 8):
        align *= 2
    bp = ((b + align - 1) // align) * align
    if bp != b:
        idx = jnp.pad(idx, ((0, bp - b), (0, 0)))
    nb = bp // nw
    nchunks = nb // rpc
    idx3 = idx.reshape(nw, nchunks, rpc * s)

    out = _gather_mean(idx3, emb_weight.astype(jnp.float32),
                       nb, nchunks, rpc, s)
    return out[:b]


# speedup vs baseline: 1.0131x; 1.0131x over previous
import functools

import jax
import jax.numpy as jnp
from jax import lax
from jax.experimental import pallas as pl
from jax.experimental.pallas import tpu as pltpu
from jax.experimental.pallas import tpu_sc as plsc

_LANES = 16
_NBUF = 3


@functools.partial(jax.jit, static_argnums=(2, 3, 4, 5))
def _gather_mean(idx3, table, nb, nchunks, rpc, s):
    info = plsc.get_sparse_core_info()
    nc, ns = info.num_cores, info.num_subcores
    nw = nc * ns
    d = table.shape[1]
    bp = nw * nb

    mesh = plsc.VectorSubcoreMesh(core_axis_name="c", subcore_axis_name="s")

    @functools.partial(
        pl.kernel,
        mesh=mesh,
        out_type=jax.ShapeDtypeStruct((bp, d), jnp.float32),
        scratch_types=[
            pltpu.VMEM((nchunks, rpc * s), jnp.int32),
        ] + [pltpu.VMEM((rpc * s, d), jnp.float32)] * _NBUF + [
            pltpu.VMEM((nb, d), jnp.float32),
        ] + [pltpu.SemaphoreType.DMA] * _NBUF,
    )
    def k(idx_hbm, table_hbm, out_hbm, idx_v, *rest):
        bufs = rest[:_NBUF]
        out_v = rest[_NBUF]
        sems = rest[_NBUF + 1:]
        w = lax.axis_index("s") * nc + lax.axis_index("c")
        pltpu.sync_copy(idx_hbm.at[w], idx_v)

        scale = jnp.float32(1.0 / s)
        nvec = d // _LANES

        def compute(c, buf):
            for r in range(rpc):
                base = jnp.int32(r * s)

                def nb_body(j, accs):
                    return tuple(
                        accs[v] + buf[base + j, pl.ds(v * _LANES, _LANES)]
                        for v in range(nvec))

                accs = tuple(buf[base, pl.ds(v * _LANES, _LANES)]
                             for v in range(nvec))
                accs = lax.fori_loop(jnp.int32(1), jnp.int32(s), nb_body,
                                     accs)
                row = c * rpc + r
                for v in range(nvec):
                    out_v[row, pl.ds(v * _LANES, _LANES)] = accs[v] * scale

        for b in range(_NBUF - 1):
            pltpu.async_copy(table_hbm.at[idx_v.at[jnp.int32(b)]],
                             bufs[b], sems[b])

        def body(t, carry):
            for b in range(_NBUF):
                c = t * _NBUF + b
                pltpu.make_async_copy(table_hbm.at[idx_v.at[c]],
                                      bufs[b], sems[b]).wait()
                nxt = c + _NBUF - 1
                nb_slot = (b + _NBUF - 1) % _NBUF

                @pl.when(nxt < nchunks)
                def _():
                    pltpu.async_copy(table_hbm.at[idx_v.at[nxt]],
                                     bufs[nb_slot], sems[nb_slot])

                compute(c, bufs[b])
            return carry

        lax.fori_loop(jnp.int32(0), jnp.int32(nchunks // _NBUF), body,
                      jnp.int32(0))
        pltpu.sync_copy(out_v, out_hbm.at[pl.ds(w * nb, nb)])

    return k(idx3, table)


def kernel(nodes, neighbors, num_sample, emb_weight):
    b, s = neighbors.shape
    idx = neighbors.astype(jnp.int32)

    info = plsc.get_sparse_core_info()
    nw = info.num_cores * info.num_subcores
    rpc = max(1, 128 // s)
    align = nw * rpc * _NBUF
    while align % (nw * 8):
        align *= 2
    bp = ((b + align - 1) // align) * align
    if bp != b:
        idx = jnp.pad(idx, ((0, bp - b), (0, 0)))
    nb = bp // nw
    nchunks = nb // rpc
    idx3 = idx.reshape(nw, nchunks, rpc * s)

    out = _gather_mean(idx3, emb_weight.astype(jnp.float32),
                       nb, nchunks, rpc, s)
    return out[:b]


# speedup vs baseline: 2.4080x; 2.3768x over previous
import functools

import jax
import jax.numpy as jnp
from jax import lax
from jax.experimental import pallas as pl
from jax.experimental.pallas import tpu as pltpu
from jax.experimental.pallas import tpu_sc as plsc

_LANES = 16
_NBUF = 2


@functools.partial(jax.jit, static_argnums=(2, 3, 4, 5))
def _gather_mean(idx3, table, nb, nchunks, rpc, s):
    info = plsc.get_sparse_core_info()
    nc, ns = info.num_cores, info.num_subcores
    nw = nc * ns
    d = table.shape[1]
    bp = nw * nb

    mesh = plsc.VectorSubcoreMesh(core_axis_name="c", subcore_axis_name="s")

    @functools.partial(
        pl.kernel,
        mesh=mesh,
        out_type=jax.ShapeDtypeStruct((bp, d), jnp.float32),
        scratch_types=[
            pltpu.VMEM((nchunks, rpc * s), jnp.int32),
        ] + [pltpu.VMEM((rpc * s, d), jnp.float32)] * _NBUF + [
            pltpu.VMEM((nb, d), jnp.float32),
        ] + [pltpu.SemaphoreType.DMA] * _NBUF,
    )
    def k(idx_hbm, table_hbm, out_hbm, idx_v, *rest):
        bufs = rest[:_NBUF]
        out_v = rest[_NBUF]
        sems = rest[_NBUF + 1:]
        w = lax.axis_index("s") * nc + lax.axis_index("c")
        pltpu.sync_copy(idx_hbm.at[w], idx_v)

        scale = jnp.float32(1.0 / s)
        nvec = d // _LANES

        def compute(c, buf):
            for r in range(rpc):
                base = jnp.int32(r * s)

                def nb_body(j, accs):
                    return tuple(
                        accs[v] + buf[base + j, pl.ds(v * _LANES, _LANES)]
                        for v in range(nvec))

                accs = tuple(buf[base, pl.ds(v * _LANES, _LANES)]
                             for v in range(nvec))
                accs = lax.fori_loop(jnp.int32(1), jnp.int32(s), nb_body,
                                     accs)
                row = c * rpc + r
                for v in range(nvec):
                    out_v[row, pl.ds(v * _LANES, _LANES)] = accs[v] * scale

        for b in range(_NBUF - 1):
            pltpu.async_copy(table_hbm.at[idx_v.at[jnp.int32(b)]],
                             bufs[b], sems[b])

        def body(t, carry):
            for b in range(_NBUF):
                c = t * _NBUF + b
                pltpu.make_async_copy(table_hbm.at[idx_v.at[c]],
                                      bufs[b], sems[b]).wait()
                nxt = c + _NBUF - 1
                nb_slot = (b + _NBUF - 1) % _NBUF

                @pl.when(nxt < nchunks)
                def _():
                    pltpu.async_copy(table_hbm.at[idx_v.at[nxt]],
                                     bufs[nb_slot], sems[nb_slot])

                compute(c, bufs[b])
            return carry

        lax.fori_loop(jnp.int32(0), jnp.int32(nchunks // _NBUF), body,
                      jnp.int32(0))
        pltpu.sync_copy(out_v, out_hbm.at[pl.ds(w * nb, nb)])

    return k(idx3, table)


def kernel(nodes, neighbors, num_sample, emb_weight):
    b, s = neighbors.shape
    idx = neighbors.astype(jnp.int32)

    info = plsc.get_sparse_core_info()
    nw = info.num_cores * info.num_subcores
    rpc = max(1, 128 // s)
    align = nw * rpc * _NBUF
    while align % (nw * 8):
        align *= 2
    bp = ((b + align - 1) // align) * align
    if bp != b:
        idx = jnp.pad(idx, ((0, bp - b), (0, 0)))
    nb = bp // nw
    nchunks = nb // rpc
    idx3 = idx.reshape(nw, nchunks, rpc * s)

    out = _gather_mean(idx3, emb_weight.astype(jnp.float32),
                       nb, nchunks, rpc, s)
    return out[:b]


# speedup vs baseline: 2.5502x; 1.0591x over previous
import functools

import jax
import jax.numpy as jnp
from jax import lax
from jax.experimental import pallas as pl
from jax.experimental.pallas import tpu as pltpu
from jax.experimental.pallas import tpu_sc as plsc

_LANES = 16
_FRAC0 = 0.75


@functools.partial(jax.jit, static_argnums=(2, 3, 4))
def _gather_mean(idx_flat, table, nb0, nb1, s):
    info = plsc.get_sparse_core_info()
    nc, ns = info.num_cores, info.num_subcores
    d = table.shape[1]
    bp = ns * (nb0 + nb1)
    rpc = 128 // s
    nch0, nch1 = nb0 // rpc, nb1 // rpc

    mesh = plsc.VectorSubcoreMesh(core_axis_name="c", subcore_axis_name="s")

    @functools.partial(
        pl.kernel,
        mesh=mesh,
        out_type=jax.ShapeDtypeStruct((bp, d), jnp.float32),
        scratch_types=[
            pltpu.VMEM((nch0 * 128,), jnp.int32),
            pltpu.VMEM((rpc * s, d), jnp.float32),
            pltpu.VMEM((rpc * s, d), jnp.float32),
            pltpu.VMEM((nb0, d), jnp.float32),
            pltpu.SemaphoreType.DMA,
            pltpu.SemaphoreType.DMA,
        ],
    )
    def k(idx_hbm, table_hbm, out_hbm, idx_v, buf0, buf1, out_v, sem0, sem1):
        cid = lax.axis_index("c")
        sid = lax.axis_index("s")
        is0 = cid == 0
        sid32 = sid.astype(jnp.int32)
        base = jnp.where(is0, sid32 * jnp.int32(nb0),
                         jnp.int32(ns * nb0) + sid32 * jnp.int32(nb1))
        nch = jnp.where(is0, jnp.int32(nch0), jnp.int32(nch1))

        pltpu.sync_copy(idx_hbm.at[pl.ds(base * s, nb1 * s)],
                        idx_v.at[pl.ds(0, nb1 * s)])

        @pl.when(is0)
        def _():
            pltpu.sync_copy(
                idx_hbm.at[pl.ds(base * s + nb1 * s, (nb0 - nb1) * s)],
                idx_v.at[pl.ds(nb1 * s, (nb0 - nb1) * s)])

        scale = jnp.float32(1.0 / s)
        nvec = d // _LANES
        bufs = (buf0, buf1)
        sems = (sem0, sem1)

        def idx_at(c):
            return idx_v.at[pl.ds(c * 128, 128)]

        def compute(c, buf):
            for r in range(rpc):
                rbase = jnp.int32(r * s)

                def nb_body(j, accs):
                    return tuple(
                        accs[v] + buf[rbase + j, pl.ds(v * _LANES, _LANES)]
                        for v in range(nvec))

                accs = tuple(buf[rbase, pl.ds(v * _LANES, _LANES)]
                             for v in range(nvec))
                accs = lax.fori_loop(jnp.int32(1), jnp.int32(s), nb_body,
                                     accs)
                row = c * rpc + r
                for v in range(nvec):
                    out_v[row, pl.ds(v * _LANES, _LANES)] = accs[v] * scale

        pltpu.async_copy(table_hbm.at[idx_at(jnp.int32(0))], buf0, sem0)

        def body(t, carry):
            for b in range(2):
                c = t * 2 + b
                pltpu.make_async_copy(table_hbm.at[idx_at(c)],
                                      bufs[b], sems[b]).wait()
                nxt = c + 1

                @pl.when(nxt < nch)
                def _():
                    pltpu.async_copy(table_hbm.at[idx_at(nxt)],
                                     bufs[1 - b], sems[1 - b])

                compute(c, bufs[b])
            return carry

        lax.fori_loop(jnp.int32(0), nch // 2, body, jnp.int32(0))

        pltpu.sync_copy(out_v.at[pl.ds(0, nb1)],
                        out_hbm.at[pl.ds(base, nb1)])

        @pl.when(is0)
        def _():
            pltpu.sync_copy(out_v.at[pl.ds(nb1, nb0 - nb1)],
                            out_hbm.at[pl.ds(base + nb1, nb0 - nb1)])

    return k(idx_flat, table)


def kernel(nodes, neighbors, num_sample, emb_weight):
    b, s = neighbors.shape
    idx = neighbors.astype(jnp.int32)

    info = plsc.get_sparse_core_info()
    ns = info.num_subcores
    rpc = max(1, 128 // s)
    grain = max(8, rpc * 2)
    align = ns * grain * 2
    bp = ((b + align - 1) // align) * align
    if bp != b:
        idx = jnp.pad(idx, ((0, bp - b), (0, 0)))
    per_pair = bp // ns
    nb0 = int(round(_FRAC0 * per_pair / grain)) * grain
    nb0 = min(max(nb0, grain), per_pair - grain)
    nb1 = per_pair - nb0

    out = _gather_mean(idx.reshape(bp * s), emb_weight.astype(jnp.float32),
                       nb0, nb1, s)
    return out[:b]


# speedup vs baseline: 2.5727x; 1.0088x over previous
import functools

import jax
import jax.numpy as jnp
from jax import lax
from jax.experimental import pallas as pl
from jax.experimental.pallas import tpu as pltpu
from jax.experimental.pallas import tpu_sc as plsc

_LANES = 16
_FRAC0 = 0.93


@functools.partial(jax.jit, static_argnums=(2, 3, 4))
def _gather_mean(idx_flat, table, nb0, nb1, s):
    info = plsc.get_sparse_core_info()
    nc, ns = info.num_cores, info.num_subcores
    d = table.shape[1]
    bp = ns * (nb0 + nb1)
    rpc = 128 // s
    nch0, nch1 = nb0 // rpc, nb1 // rpc

    mesh = plsc.VectorSubcoreMesh(core_axis_name="c", subcore_axis_name="s")

    @functools.partial(
        pl.kernel,
        mesh=mesh,
        out_type=jax.ShapeDtypeStruct((bp, d), jnp.float32),
        scratch_types=[
            pltpu.VMEM((nch0 * 128,), jnp.int32),
            pltpu.VMEM((rpc * s, d), jnp.float32),
            pltpu.VMEM((rpc * s, d), jnp.float32),
            pltpu.VMEM((nb0, d), jnp.float32),
            pltpu.SemaphoreType.DMA,
            pltpu.SemaphoreType.DMA,
        ],
    )
    def k(idx_hbm, table_hbm, out_hbm, idx_v, buf0, buf1, out_v, sem0, sem1):
        cid = lax.axis_index("c")
        sid = lax.axis_index("s")
        is0 = cid == 0
        sid32 = sid.astype(jnp.int32)
        base = jnp.where(is0, sid32 * jnp.int32(nb0),
                         jnp.int32(ns * nb0) + sid32 * jnp.int32(nb1))
        nch = jnp.where(is0, jnp.int32(nch0), jnp.int32(nch1))

        pltpu.sync_copy(idx_hbm.at[pl.ds(base * s, nb1 * s)],
                        idx_v.at[pl.ds(0, nb1 * s)])

        @pl.when(is0)
        def _():
            pltpu.sync_copy(
                idx_hbm.at[pl.ds(base * s + nb1 * s, (nb0 - nb1) * s)],
                idx_v.at[pl.ds(nb1 * s, (nb0 - nb1) * s)])

        scale = jnp.float32(1.0 / s)
        nvec = d // _LANES
        bufs = (buf0, buf1)
        sems = (sem0, sem1)

        def idx_at(c):
            return idx_v.at[pl.ds(c * 128, 128)]

        def compute(c, buf):
            for r in range(rpc):
                rbase = jnp.int32(r * s)

                def nb_body(j, accs):
                    return tuple(
                        accs[v] + buf[rbase + j, pl.ds(v * _LANES, _LANES)]
                        for v in range(nvec))

                accs = tuple(buf[rbase, pl.ds(v * _LANES, _LANES)]
                             for v in range(nvec))
                accs = lax.fori_loop(jnp.int32(1), jnp.int32(s), nb_body,
                                     accs)
                row = c * rpc + r
                for v in range(nvec):
                    out_v[row, pl.ds(v * _LANES, _LANES)] = accs[v] * scale

        pltpu.async_copy(table_hbm.at[idx_at(jnp.int32(0))], buf0, sem0)

        def body(t, carry):
            for b in range(2):
                c = t * 2 + b
                pltpu.make_async_copy(table_hbm.at[idx_at(c)],
                                      bufs[b], sems[b]).wait()
                nxt = c + 1

                @pl.when(nxt < nch)
                def _():
                    pltpu.async_copy(table_hbm.at[idx_at(nxt)],
                                     bufs[1 - b], sems[1 - b])

                compute(c, bufs[b])
            return carry

        lax.fori_loop(jnp.int32(0), nch // 2, body, jnp.int32(0))

        pltpu.sync_copy(out_v.at[pl.ds(0, nb1)],
                        out_hbm.at[pl.ds(base, nb1)])

        @pl.when(is0)
        def _():
            pltpu.sync_copy(out_v.at[pl.ds(nb1, nb0 - nb1)],
                            out_hbm.at[pl.ds(base + nb1, nb0 - nb1)])

    return k(idx_flat, table)


def kernel(nodes, neighbors, num_sample, emb_weight):
    b, s = neighbors.shape
    idx = neighbors.astype(jnp.int32)

    info = plsc.get_sparse_core_info()
    ns = info.num_subcores
    rpc = max(1, 128 // s)
    grain = max(8, rpc * 2)
    align = ns * grain * 2
    bp = ((b + align - 1) // align) * align
    if bp != b:
        idx = jnp.pad(idx, ((0, bp - b), (0, 0)))
    per_pair = bp // ns
    nb0 = int(round(_FRAC0 * per_pair / grain)) * grain
    nb0 = min(max(nb0, grain), per_pair - grain)
    nb1 = per_pair - nb0

    out = _gather_mean(idx.reshape(bp * s), emb_weight.astype(jnp.float32),
                       nb0, nb1, s)
    return out[:b]
